# Initial kernel scaffold; baseline (speedup 1.0000x reference)
import jax
import jax.numpy as jnp
from jax.experimental import pallas as pl


def kernel(matrix_encodings, edge_index, batch, W1l, b1l, W1r, W2l, b2l, W2r, Wlin, blin):
    raise NotImplementedError("write your pallas kernel here")


# speedup vs baseline: 20.7865x; 20.7865x over previous
import functools

import jax
import jax.numpy as jnp
from jax import lax
from jax.experimental import pallas as pl
from jax.experimental.pallas import tpu as pltpu
from jax.experimental.pallas import tpu_sc as plsc

_N = 100000
_E = 3200000
_G = 1000
_D = 16

_NC = 2
_NS = 16
_NW = _NC * _NS

_CH = 128
_T = 8
_OUTER = 99
_CPS = _T * _OUTER
_EPAD = _NW * _CPS * _CH
_NCHUNK = _EPAD // _CH

_R = 100096
_RPS = _R // _NS
_ZC = _RPS // 4
_ZW = _RPS // 2

_BN = 2176

_RP = 102400
_NPCH = _RP // _CH
_PPS = _NPCH // _NS
_GP = 1024
_GPS = _GP // _NS


def _mesh():
    return plsc.VectorSubcoreMesh(core_axis_name="c", subcore_axis_name="s")


def _make_edge_pass(with_cnt: bool):
    out_type = [jax.ShapeDtypeStruct((_NC, _R, _D), jnp.float32)]
    if with_cnt:
        out_type.append(jax.ShapeDtypeStruct((_NC, _R), jnp.float32))
    scratch = [
        pltpu.VMEM((_T, _CH), jnp.int32),
        pltpu.VMEM((_T, _CH), jnp.int32),
        pltpu.VMEM((_T * _CH, _D), jnp.float32),
        pltpu.VMEM_SHARED((_R, _D), jnp.float32),
    ]
    if with_cnt:
        scratch += [
            pltpu.VMEM((_CH,), jnp.float32),
            pltpu.VMEM((_ZW,), jnp.float32),
            pltpu.VMEM_SHARED((_R,), jnp.float32),
        ]
    scratch += [
        pltpu.SemaphoreType.DMA,
        pltpu.SemaphoreType.DMA,
        pltpu.SemaphoreType.DMA,
    ]

    @functools.partial(
        pl.kernel, mesh=_mesh(), out_type=out_type, scratch_types=scratch,
        compiler_params=pltpu.CompilerParams(use_tc_tiling_on_sc=False))
    def edge_pass(table, srcg, dstg, *rest):
        if with_cnt:
            (acc_out, cnt_out, idx_s, idx_d, rows, acc_sh, ones, zbuf,
             cnt_sh, gsem, ssem, csem) = rest
        else:
            (acc_out, idx_s, idx_d, rows, acc_sh, gsem, ssem, csem) = rest
        c = lax.axis_index("c")
        s = lax.axis_index("s")
        w = c * _NS + s

        @pl.loop(0, _ZC)
        def _(i):
            rows[i, :] = jnp.zeros((_D,), jnp.float32)

        for k in range(4):
            pltpu.sync_copy(rows.at[pl.ds(0, _ZC)],
                            acc_sh.at[pl.ds(s * _RPS + k * _ZC, _ZC)])
        if with_cnt:
            @pl.loop(0, _ZW // 16)
            def _(i):
                zbuf[pl.ds(i * 16, 16)] = jnp.zeros((16,), jnp.float32)

            for k in range(2):
                pltpu.sync_copy(zbuf,
                                cnt_sh.at[pl.ds(s * _RPS + k * _ZW, _ZW)])

            @pl.loop(0, _CH // 16)
            def _(i):
                ones[pl.ds(i * 16, 16)] = jnp.ones((16,), jnp.float32)

        plsc.subcore_barrier()

        base = w * _CPS

        @pl.loop(0, _OUTER)
        def _(it):
            c0 = base + it * _T
            pltpu.sync_copy(srcg.at[pl.ds(c0, _T)], idx_s)
            pltpu.sync_copy(dstg.at[pl.ds(c0, _T)], idx_d)
            hs = [pltpu.async_copy(table.at[idx_s.at[j]],
                                   rows.at[pl.ds(j * _CH, _CH)], gsem)
                  for j in range(_T)]
            for h in hs:
                h.wait()
            ha = [pltpu.async_copy(rows.at[pl.ds(j * _CH, _CH)],
                                   acc_sh.at[idx_d.at[j]], ssem, add=True)
                  for j in range(_T)]
            if with_cnt:
                hc = [pltpu.async_copy(ones, cnt_sh.at[idx_d.at[j]], csem,
                                       add=True)
                      for j in range(_T)]
            for h in ha:
                h.wait()
            if with_cnt:
                for h in hc:
                    h.wait()

        plsc.subcore_barrier()

        r0 = s * _RPS
        pltpu.sync_copy(acc_sh.at[pl.ds(r0, _RPS)],
                        acc_out.at[c, pl.ds(r0, _RPS)])
        if with_cnt:
            pltpu.sync_copy(cnt_sh.at[pl.ds(r0, _RPS)],
                            cnt_out.at[c, pl.ds(r0, _RPS)])

    return edge_pass


_edge_pass_cnt = _make_edge_pass(True)
_edge_pass = _make_edge_pass(False)


@functools.partial(pl.kernel, mesh=_mesh(),
                   out_type=jax.ShapeDtypeStruct((_GP,), jnp.float32),
                   compiler_params=pltpu.CompilerParams(
                       use_tc_tiling_on_sc=False),
                   scratch_types=[
                       pltpu.VMEM((1, _CH), jnp.float32),
                       pltpu.VMEM((1, _CH), jnp.int32),
                       pltpu.VMEM((_CH,), jnp.float32),
                       pltpu.VMEM((_GPS,), jnp.float32),
                       pltpu.VMEM((_GPS,), jnp.float32),
                       pltpu.VMEM((16,), jnp.float32),
                       pltpu.VMEM((_GPS,), jnp.float32),
                       pltpu.VMEM_SHARED((_GP,), jnp.float32),
                       pltpu.VMEM_SHARED((_GP,), jnp.float32),
                   ])
def _pool_pass(tg, bg, blin_h, out, tbuf, ibuf, ones, pbuf, cbuf, bbuf, obuf,
               psum_sh, pcnt_sh):
    c = lax.axis_index("c")
    s = lax.axis_index("s")

    @pl.when(c == 0)
    def _():
        @pl.loop(0, _CH // 16)
        def _(i):
            ones[pl.ds(i * 16, 16)] = jnp.ones((16,), jnp.float32)

        @pl.loop(0, _GPS // 16)
        def _(i):
            pbuf[pl.ds(i * 16, 16)] = jnp.zeros((16,), jnp.float32)

        pltpu.sync_copy(pbuf, psum_sh.at[pl.ds(s * _GPS, _GPS)])
        pltpu.sync_copy(pbuf, pcnt_sh.at[pl.ds(s * _GPS, _GPS)])
        plsc.subcore_barrier()

        @pl.loop(0, _PPS)
        def _(it):
            ch = s * _PPS + it
            pltpu.sync_copy(tg.at[ch], tbuf)
            pltpu.sync_copy(bg.at[ch], ibuf)
            pltpu.sync_copy(tbuf.at[0], psum_sh.at[ibuf.at[0]], add=True)
            pltpu.sync_copy(ones, pcnt_sh.at[ibuf.at[0]], add=True)

        plsc.subcore_barrier()

        pltpu.sync_copy(psum_sh.at[pl.ds(s * _GPS, _GPS)], pbuf)
        pltpu.sync_copy(pcnt_sh.at[pl.ds(s * _GPS, _GPS)], cbuf)
        pltpu.sync_copy(blin_h, bbuf)
        b = bbuf[pl.ds(0, 16)][0]
        for k in range(_GPS // 16):
            obuf[pl.ds(k * 16, 16)] = (
                pbuf[pl.ds(k * 16, 16)]
                / jnp.maximum(cbuf[pl.ds(k * 16, 16)], 1.0) + b)
        pltpu.sync_copy(obuf, out.at[pl.ds(s * _GPS, _GPS)])


def _p1_body(x_ref, w_ref, o_ref):
    o_ref[:] = jnp.dot(x_ref[:], w_ref[:], preferred_element_type=jnp.float32)


def _qp_body(acc_ref, c0_ref, c1_ref, x_ref, w1rt_ref, b1_ref, w2lt_ref,
             w2rt_ref, b2_ref, a2_ref, r2_ref):
    cnt = jnp.maximum(c0_ref[:] + c1_ref[:], 1.0)
    mean = (acc_ref[0] + acc_ref[1]) / cnt
    h1 = mean + jnp.dot(x_ref[:], w1rt_ref[:],
                        preferred_element_type=jnp.float32) + b1_ref[:]
    h1 = jnp.maximum(h1, 0.0)
    a2_ref[:] = jnp.dot(h1, w2lt_ref[:], preferred_element_type=jnp.float32)
    r2_ref[:] = jnp.dot(h1, w2rt_ref[:],
                        preferred_element_type=jnp.float32) + b2_ref[:]


def _q2_body(acc_ref, c0_ref, c1_ref, r2_ref, wlin_ref, t_ref):
    cnt = jnp.maximum(c0_ref[:] + c1_ref[:], 1.0)
    h2 = (acc_ref[0] + acc_ref[1]) / cnt + r2_ref[:]
    t_ref[:] = jnp.sum(h2 * wlin_ref[:], axis=1, keepdims=True)


def _row_spec():
    return pl.BlockSpec((_BN, _D), lambda i: (i, 0))


def _vec_spec():
    return pl.BlockSpec((_BN, 1), lambda i: (i, 0))


def _w_spec():
    return pl.BlockSpec((_D, _D), lambda i: (0, 0))


def _b_spec():
    return pl.BlockSpec((1, _D), lambda i: (0, 0))


def _acc_spec():
    return pl.BlockSpec((_NC, _BN, _D), lambda i: (0, i, 0))


def kernel(matrix_encodings, edge_index, batch, W1l, b1l, W1r, W2l, b2l, W2r,
           Wlin, blin):
    x = matrix_encodings
    f32, i32 = jnp.float32, jnp.int32
    grid = (_R // _BN,)

    srcg = jnp.concatenate(
        [edge_index[0], jnp.zeros((_EPAD - _E,), i32)]).reshape(_NCHUNK, _CH)
    dstg = jnp.concatenate(
        [edge_index[1], jnp.full((_EPAD - _E,), _N, i32)]).reshape(_NCHUNK, _CH)
    x_pad = jnp.pad(x, ((0, _R - _N), (0, 0)))

    a1 = pl.pallas_call(
        _p1_body, grid=grid,
        in_specs=[_row_spec(), _w_spec()],
        out_specs=_row_spec(),
        out_shape=jax.ShapeDtypeStruct((_R, _D), f32),
    )(x_pad, W1l.T)

    acc1, cnt = _edge_pass_cnt(a1, srcg, dstg)

    a2, r2 = pl.pallas_call(
        _qp_body, grid=grid,
        in_specs=[_acc_spec(), _vec_spec(), _vec_spec(), _row_spec(),
                  _w_spec(), _b_spec(), _w_spec(), _w_spec(), _b_spec()],
        out_specs=[_row_spec(), _row_spec()],
        out_shape=[jax.ShapeDtypeStruct((_R, _D), f32),
                   jax.ShapeDtypeStruct((_R, _D), f32)],
    )(acc1, cnt[0].reshape(_R, 1), cnt[1].reshape(_R, 1), x_pad, W1r.T,
      b1l.reshape(1, _D), W2l.T, W2r.T, b2l.reshape(1, _D))

    (acc2,) = _edge_pass(a2, srcg, dstg)

    t = pl.pallas_call(
        _q2_body, grid=grid,
        in_specs=[_acc_spec(), _vec_spec(), _vec_spec(), _row_spec(),
                  _b_spec()],
        out_specs=_vec_spec(),
        out_shape=jax.ShapeDtypeStruct((_R, 1), f32),
    )(acc2, cnt[0].reshape(_R, 1), cnt[1].reshape(_R, 1), r2, Wlin)

    tg = jnp.concatenate(
        [t.reshape(_R), jnp.zeros((_RP - _R,), f32)]).reshape(_NPCH, 1, _CH)
    bg = jnp.concatenate(
        [batch, jnp.full((_RP - _N,), _G, i32)]).reshape(_NPCH, 1, _CH)
    pooled = _pool_pass(tg, bg, jnp.pad(blin, (0, 15)))
    return pooled[:_G].reshape(_G, 1)


# speedup vs baseline: 23.8792x; 1.1488x over previous
import functools

import jax
import jax.numpy as jnp
from jax import lax
from jax.experimental import pallas as pl
from jax.experimental.pallas import tpu as pltpu
from jax.experimental.pallas import tpu_sc as plsc

_N = 100000
_E = 3200000
_G = 1000
_D = 16

_NC = 2
_NS = 16
_NW = _NC * _NS

_CH = 128
_T = 8
_OUTER = 99
_CPS = _T * _OUTER
_EPAD = _NW * _CPS * _CH
_NCHUNK = _EPAD // _CH

_R = 100096
_RPS = _R // _NS
_ZC = _RPS // 4
_ZW = _RPS // 2

_BN = 4000

_RP = 102400
_NPCH = _RP // _CH
_PPS = _NPCH // _NS
_GP = 1024
_GPS = _GP // _NS


def _mesh():
    return plsc.VectorSubcoreMesh(core_axis_name="c", subcore_axis_name="s")


def _make_edge_pass(with_cnt: bool):
    out_type = [jax.ShapeDtypeStruct((_NC, _R, _D), jnp.float32)]
    if with_cnt:
        out_type.append(jax.ShapeDtypeStruct((_NC, _R), jnp.float32))
    scratch = [
        pltpu.VMEM((_T, _CH), jnp.int32),
        pltpu.VMEM((_T, _CH), jnp.int32),
        pltpu.VMEM((_T * _CH, _D), jnp.float32),
        pltpu.VMEM_SHARED((_R, _D), jnp.float32),
    ]
    if with_cnt:
        scratch += [
            pltpu.VMEM((_CH,), jnp.float32),
            pltpu.VMEM((_ZW,), jnp.float32),
            pltpu.VMEM_SHARED((_R,), jnp.float32),
        ]
    scratch += [
        pltpu.SemaphoreType.DMA,
        pltpu.SemaphoreType.DMA,
        pltpu.SemaphoreType.DMA,
    ]

    @functools.partial(
        pl.kernel, mesh=_mesh(), out_type=out_type, scratch_types=scratch,
        compiler_params=pltpu.CompilerParams(use_tc_tiling_on_sc=False))
    def edge_pass(table, srcg, dstg, *rest):
        if with_cnt:
            (acc_out, cnt_out, idx_s, idx_d, rows, acc_sh, ones, zbuf,
             cnt_sh, gsem, ssem, csem) = rest
        else:
            (acc_out, idx_s, idx_d, rows, acc_sh, gsem, ssem, csem) = rest
        c = lax.axis_index("c")
        s = lax.axis_index("s")
        w = c * _NS + s

        @pl.loop(0, _ZC)
        def _(i):
            rows[i, :] = jnp.zeros((_D,), jnp.float32)

        for k in range(4):
            pltpu.sync_copy(rows.at[pl.ds(0, _ZC)],
                            acc_sh.at[pl.ds(s * _RPS + k * _ZC, _ZC)])
        if with_cnt:
            @pl.loop(0, _ZW // 16)
            def _(i):
                zbuf[pl.ds(i * 16, 16)] = jnp.zeros((16,), jnp.float32)

            for k in range(2):
                pltpu.sync_copy(zbuf,
                                cnt_sh.at[pl.ds(s * _RPS + k * _ZW, _ZW)])

            @pl.loop(0, _CH // 16)
            def _(i):
                ones[pl.ds(i * 16, 16)] = jnp.ones((16,), jnp.float32)

        plsc.subcore_barrier()

        base = w * _CPS

        @pl.loop(0, _OUTER)
        def _(it):
            c0 = base + it * _T
            pltpu.sync_copy(srcg.at[pl.ds(c0, _T)], idx_s)
            pltpu.sync_copy(dstg.at[pl.ds(c0, _T)], idx_d)
            hs = [pltpu.async_copy(table.at[idx_s.at[j]],
                                   rows.at[pl.ds(j * _CH, _CH)], gsem)
                  for j in range(_T)]
            for h in hs:
                h.wait()
            ha = [pltpu.async_copy(rows.at[pl.ds(j * _CH, _CH)],
                                   acc_sh.at[idx_d.at[j]], ssem, add=True)
                  for j in range(_T)]
            if with_cnt:
                hc = [pltpu.async_copy(ones, cnt_sh.at[idx_d.at[j]], csem,
                                       add=True)
                      for j in range(_T)]
            for h in ha:
                h.wait()
            if with_cnt:
                for h in hc:
                    h.wait()

        plsc.subcore_barrier()

        r0 = s * _RPS
        pltpu.sync_copy(acc_sh.at[pl.ds(r0, _RPS)],
                        acc_out.at[c, pl.ds(r0, _RPS)])
        if with_cnt:
            pltpu.sync_copy(cnt_sh.at[pl.ds(r0, _RPS)],
                            cnt_out.at[c, pl.ds(r0, _RPS)])

    return edge_pass


_edge_pass_cnt = _make_edge_pass(True)
_edge_pass = _make_edge_pass(False)


@functools.partial(pl.kernel, mesh=_mesh(),
                   out_type=jax.ShapeDtypeStruct((_GP,), jnp.float32),
                   compiler_params=pltpu.CompilerParams(
                       use_tc_tiling_on_sc=False),
                   scratch_types=[
                       pltpu.VMEM((1, _CH), jnp.float32),
                       pltpu.VMEM((1, _CH), jnp.int32),
                       pltpu.VMEM((_CH,), jnp.float32),
                       pltpu.VMEM((_GPS,), jnp.float32),
                       pltpu.VMEM((_GPS,), jnp.float32),
                       pltpu.VMEM((16,), jnp.float32),
                       pltpu.VMEM((_GPS,), jnp.float32),
                       pltpu.VMEM_SHARED((_GP,), jnp.float32),
                       pltpu.VMEM_SHARED((_GP,), jnp.float32),
                   ])
def _pool_pass(tg, bg, blin_h, out, tbuf, ibuf, ones, pbuf, cbuf, bbuf, obuf,
               psum_sh, pcnt_sh):
    c = lax.axis_index("c")
    s = lax.axis_index("s")

    @pl.when(c == 0)
    def _():
        @pl.loop(0, _CH // 16)
        def _(i):
            ones[pl.ds(i * 16, 16)] = jnp.ones((16,), jnp.float32)

        @pl.loop(0, _GPS // 16)
        def _(i):
            pbuf[pl.ds(i * 16, 16)] = jnp.zeros((16,), jnp.float32)

        pltpu.sync_copy(pbuf, psum_sh.at[pl.ds(s * _GPS, _GPS)])
        pltpu.sync_copy(pbuf, pcnt_sh.at[pl.ds(s * _GPS, _GPS)])
        plsc.subcore_barrier()

        @pl.loop(0, _PPS)
        def _(it):
            ch = s * _PPS + it
            pltpu.sync_copy(tg.at[ch], tbuf)
            pltpu.sync_copy(bg.at[ch], ibuf)
            pltpu.sync_copy(tbuf.at[0], psum_sh.at[ibuf.at[0]], add=True)
            pltpu.sync_copy(ones, pcnt_sh.at[ibuf.at[0]], add=True)

        plsc.subcore_barrier()

        pltpu.sync_copy(psum_sh.at[pl.ds(s * _GPS, _GPS)], pbuf)
        pltpu.sync_copy(pcnt_sh.at[pl.ds(s * _GPS, _GPS)], cbuf)
        pltpu.sync_copy(blin_h, bbuf)
        b = bbuf[pl.ds(0, 16)][0]
        for k in range(_GPS // 16):
            obuf[pl.ds(k * 16, 16)] = (
                pbuf[pl.ds(k * 16, 16)]
                / jnp.maximum(cbuf[pl.ds(k * 16, 16)], 1.0) + b)
        pltpu.sync_copy(obuf, out.at[pl.ds(s * _GPS, _GPS)])


def _p1_body(x_ref, w_ref, o_ref):
    o_ref[:] = jnp.dot(x_ref[:], w_ref[:], preferred_element_type=jnp.float32)


def _qp_body(acc_ref, c0_ref, c1_ref, x_ref, w1rt_ref, b1_ref, w2lt_ref,
             w2rt_ref, b2_ref, a2_ref, r2_ref):
    cnt = jnp.maximum(c0_ref[:] + c1_ref[:], 1.0)
    mean = (acc_ref[0] + acc_ref[1]) / cnt
    h1 = mean + jnp.dot(x_ref[:], w1rt_ref[:],
                        preferred_element_type=jnp.float32) + b1_ref[:]
    h1 = jnp.maximum(h1, 0.0)
    a2_ref[:] = jnp.dot(h1, w2lt_ref[:], preferred_element_type=jnp.float32)
    r2_ref[:] = jnp.dot(h1, w2rt_ref[:],
                        preferred_element_type=jnp.float32) + b2_ref[:]


def _q2_body(acc_ref, c0_ref, c1_ref, r2_ref, wlin_ref, t_ref):
    cnt = jnp.maximum(c0_ref[:] + c1_ref[:], 1.0)
    h2 = (acc_ref[0] + acc_ref[1]) / cnt + r2_ref[:]
    t_ref[:] = jnp.sum(h2 * wlin_ref[:], axis=1, keepdims=True)


def _row_spec():
    return pl.BlockSpec((_BN, _D), lambda i: (i, 0))


def _vec_spec():
    return pl.BlockSpec((_BN, 1), lambda i: (i, 0))


def _w_spec():
    return pl.BlockSpec((_D, _D), lambda i: (0, 0))


def _b_spec():
    return pl.BlockSpec((1, _D), lambda i: (0, 0))


def _acc_spec():
    return pl.BlockSpec((_NC, _BN, _D), lambda i: (0, i, 0))


def kernel(matrix_encodings, edge_index, batch, W1l, b1l, W1r, W2l, b2l, W2r,
           Wlin, blin):
    x = matrix_encodings
    f32, i32 = jnp.float32, jnp.int32
    grid = (_N // _BN,)

    srcg = jnp.concatenate(
        [edge_index[0], jnp.zeros((_EPAD - _E,), i32)]).reshape(_NCHUNK, _CH)
    dstg = jnp.concatenate(
        [edge_index[1],
         _N + (jnp.arange(_EPAD - _E, dtype=i32) % (_R - _N))]
    ).reshape(_NCHUNK, _CH)

    a1 = pl.pallas_call(
        _p1_body, grid=grid,
        in_specs=[_row_spec(), _w_spec()],
        out_specs=_row_spec(),
        out_shape=jax.ShapeDtypeStruct((_N, _D), f32),
    )(x, W1l.T)

    acc1, cnt = _edge_pass_cnt(a1, srcg, dstg)

    a2, r2 = pl.pallas_call(
        _qp_body, grid=grid,
        in_specs=[_acc_spec(), _vec_spec(), _vec_spec(), _row_spec(),
                  _w_spec(), _b_spec(), _w_spec(), _w_spec(), _b_spec()],
        out_specs=[_row_spec(), _row_spec()],
        out_shape=[jax.ShapeDtypeStruct((_N, _D), f32),
                   jax.ShapeDtypeStruct((_N, _D), f32)],
    )(acc1, cnt[0].reshape(_R, 1), cnt[1].reshape(_R, 1), x, W1r.T,
      b1l.reshape(1, _D), W2l.T, W2r.T, b2l.reshape(1, _D))

    (acc2,) = _edge_pass(a2, srcg, dstg)

    t = pl.pallas_call(
        _q2_body, grid=grid,
        in_specs=[_acc_spec(), _vec_spec(), _vec_spec(), _row_spec(),
                  _b_spec()],
        out_specs=_vec_spec(),
        out_shape=jax.ShapeDtypeStruct((_N, 1), f32),
    )(acc2, cnt[0].reshape(_R, 1), cnt[1].reshape(_R, 1), r2, Wlin)

    tg = jnp.concatenate(
        [t.reshape(_N), jnp.zeros((_RP - _N,), f32)]).reshape(_NPCH, 1, _CH)
    bg = jnp.concatenate(
        [batch, jnp.full((_RP - _N,), _G, i32)]).reshape(_NPCH, 1, _CH)
    pooled = _pool_pass(tg, bg, jnp.pad(blin, (0, 15)))
    return pooled[:_G].reshape(_G, 1)
